# baseline (speedup 1.0000x reference)
import functools

import jax
import jax.numpy as jnp
from jax import lax
from jax.experimental import pallas as pl
from jax.experimental.pallas import tpu as pltpu
from jax.experimental.pallas import tpu_sc as plsc

_B = 16384
_NF = 26
_V = 100000
_EMB = 16
_NW = 32
_IDX_ROWS = _B * _NF // 128
_ROWS_PW = _IDX_ROWS // _NW
_G = 8
_CHUNKS = _ROWS_PW // _G
_CH_ROWS = _G * 128
_BM = 512


def _sc_gather(flat_idx, tab, wtab):
    mesh = plsc.VectorSubcoreMesh(core_axis_name="c", subcore_axis_name="s")

    @functools.partial(
        pl.kernel,
        mesh=mesh,
        compiler_params=pltpu.CompilerParams(use_tc_tiling_on_sc=False),
        out_type=[
            jax.ShapeDtypeStruct((_B * _NF, _EMB), jnp.float32),
            jax.ShapeDtypeStruct((_B * _NF,), jnp.float32),
        ],
        scratch_types=[
            pltpu.VMEM((_ROWS_PW, 128), jnp.int32),
            pltpu.VMEM((_CH_ROWS, _EMB), jnp.float32),
            pltpu.VMEM((_CH_ROWS,), jnp.float32),
            pltpu.SemaphoreType.DMA,
            pltpu.SemaphoreType.DMA,
        ],
    )
    def k(idx_hbm, tab_hbm, wtab_hbm, emb_out, w_out, idx_v, rows_v, wrows_v,
          sem, wsem):
        wid = lax.axis_index("s") * 2 + lax.axis_index("c")
        row0 = wid * _ROWS_PW
        pltpu.sync_copy(idx_hbm.at[pl.ds(row0, _ROWS_PW)], idx_v)

        def chunk(sc, carry):
            handles = []
            for j in range(_G):
                r = sc * _G + j
                handles.append(pltpu.async_copy(
                    tab_hbm.at[idx_v.at[r]],
                    rows_v.at[pl.ds(j * 128, 128)], sem))
                handles.append(pltpu.async_copy(
                    wtab_hbm.at[idx_v.at[r]],
                    wrows_v.at[pl.ds(j * 128, 128)], wsem))
            for h in handles:
                h.wait()
            o0 = row0 * 128 + sc * _CH_ROWS
            pltpu.sync_copy(rows_v, emb_out.at[pl.ds(o0, _CH_ROWS)])
            pltpu.sync_copy(wrows_v, w_out.at[pl.ds(o0, _CH_ROWS)])
            return carry

        lax.fori_loop(0, _CHUNKS, chunk, 0)

    return k(flat_idx, tab, wtab)


def _tc_head_body(emb_ref, da_ref, w_ref, W1a_ref, W1c_ref, b1_ref, W2_ref,
                  b2_ref, W3_ref, fmW_ref, b3_ref, M_ref, out_ref):
    emb = emb_ref[...]
    da = da_ref[...]
    h = (jnp.dot(emb, W1a_ref[...], preferred_element_type=jnp.float32)
         + jnp.dot(da, W1c_ref[...], preferred_element_type=jnp.float32)
         + b1_ref[...])
    h = jnp.maximum(h, 0.0)
    h = (jnp.dot(h, W2_ref[...], preferred_element_type=jnp.float32)
         + b2_ref[...])
    h = jnp.maximum(h, 0.0)
    deep = jnp.dot(h, W3_ref[...], preferred_element_type=jnp.float32)
    fm1 = (jnp.sum(w_ref[...], axis=1, keepdims=True)
           + jnp.dot(da, fmW_ref[...], preferred_element_type=jnp.float32))
    s = jnp.dot(emb, M_ref[...], preferred_element_type=jnp.float32)
    fm2 = 0.5 * (jnp.sum(s * s, axis=1, keepdims=True)
                 - jnp.sum(emb * emb, axis=1, keepdims=True))
    out_ref[...] = jax.nn.sigmoid(deep + fm1 + fm2 + b3_ref[...])


def _tc_head(emb2, da, wf, W1a, W1c, b1s, W2s, b2s, W3, fmW, bias3, M,
             interpret=False):
    grid = (_B // _BM,)

    def full(a):
        return pl.BlockSpec(a.shape, lambda i: (0, 0))

    return pl.pallas_call(
        _tc_head_body,
        grid=grid,
        in_specs=[
            pl.BlockSpec((_BM, _NF * _EMB), lambda i: (i, 0)),
            pl.BlockSpec((_BM, da.shape[1]), lambda i: (i, 0)),
            pl.BlockSpec((_BM, _NF), lambda i: (i, 0)),
            full(W1a), full(W1c), full(b1s), full(W2s), full(b2s),
            full(W3), full(fmW), full(bias3), full(M),
        ],
        out_specs=pl.BlockSpec((_BM, 1), lambda i: (i, 0)),
        out_shape=jax.ShapeDtypeStruct((_B, 1), jnp.float32),
        interpret=interpret,
    )(emb2, da, wf, W1a, W1c, b1s, W2s, b2s, W3, fmW, bias3, M)


def kernel(dense_inputs, sparse_inputs, bge_inputs, emb_tables, w_tables,
           fm_dense_W, fm_dense_b, W1, b1, g1, be1, W2, b2, g2, be2, W3, b3):
    tab = emb_tables.reshape(_NF * _V, _EMB)
    wtab = w_tables.reshape(_NF * _V)
    offs = (jnp.arange(_NF, dtype=jnp.int32) * _V)[None, :]
    flat_idx = (sparse_inputs.astype(jnp.int32) + offs).reshape(_IDX_ROWS, 128)

    emb_flat, w_flat = _sc_gather(flat_idx, tab, wtab)
    emb2 = emb_flat.reshape(_B, _NF * _EMB)
    wf = w_flat.reshape(_B, _NF)

    da = jnp.concatenate([dense_inputs, bge_inputs], axis=1)
    inv = 1.0 / jnp.sqrt(jnp.float32(1.0 + 1e-5))
    sc1 = g1 * inv
    W1s = W1 * sc1[None, :]
    b1s = (b1 * sc1 + be1)[None, :]
    sc2 = g2 * inv
    W2s = W2 * sc2[None, :]
    b2s = (b2 * sc2 + be2)[None, :]
    W1a = W1s[: _NF * _EMB]
    W1c = W1s[_NF * _EMB:]
    bias3 = (b3 + fm_dense_b).reshape(1, 1)
    M = jnp.tile(jnp.eye(_EMB, dtype=jnp.float32), (_NF, 1))

    return _tc_head(emb2, da, wf, W1a, W1c, b1s, W2s, b2s, W3, fmW=fm_dense_W,
                    bias3=bias3, M=M)


# speedup vs baseline: 1.4491x; 1.4491x over previous
import functools

import jax
import jax.numpy as jnp
from jax import lax
from jax.experimental import pallas as pl
from jax.experimental.pallas import tpu as pltpu
from jax.experimental.pallas import tpu_sc as plsc

_B = 16384
_NF = 26
_V = 100000
_EMB = 16
_NC = _NF * _EMB
_NW = 32
_CPW = _NC // _NW
_JROWS = _B // 128
_BM = 512


def _sc_gather(sparse_t3, tab_f, wtab_f):
    mesh = plsc.VectorSubcoreMesh(core_axis_name="c", subcore_axis_name="s")

    @functools.partial(
        pl.kernel,
        mesh=mesh,
        compiler_params=pltpu.CompilerParams(use_tc_tiling_on_sc=False),
        out_type=[
            jax.ShapeDtypeStruct((_NC, _B), jnp.float32),
            jax.ShapeDtypeStruct((_NF, _B), jnp.float32),
        ],
        scratch_types=[
            pltpu.VMEM((2, _JROWS, 128), jnp.int32),
            pltpu.VMEM((_B,), jnp.float32),
            pltpu.SemaphoreType.DMA,
        ],
    )
    def k(sp_hbm, tab_hbm, wtab_hbm, embt_out, wt_out, idx_v, col_v, sem):
        wid = lax.axis_index("s") * 2 + lax.axis_index("c")
        c0 = wid * _CPW
        f0 = lax.shift_right_logical(c0, 4)
        f_last = lax.shift_right_logical(c0 + (_CPW - 1), 4)
        pltpu.sync_copy(sp_hbm.at[f0], idx_v.at[0])
        pltpu.sync_copy(sp_hbm.at[f_last], idx_v.at[1])

        def gather_col(src_ref, base, idx_sel, out_row_ref):
            def grp(s, carry):
                hs = []
                for kk in range(16):
                    j = s * 16 + kk
                    hs.append(pltpu.async_copy(
                        src_ref.at[pl.ds(base, _V)].at[idx_v.at[idx_sel, j]],
                        col_v.at[pl.ds(j * 128, 128)], sem))
                for h in hs:
                    h.wait()
                return carry

            lax.fori_loop(0, _JROWS // 16, grp, 0)
            pltpu.sync_copy(col_v, out_row_ref)

        for t in range(_CPW):
            c = c0 + t
            f = lax.shift_right_logical(c, 4)
            sel = f - f0
            gather_col(tab_hbm, c * _V, sel, embt_out.at[c])

        @pl.when(wid < _NF)
        def _():
            pltpu.sync_copy(sp_hbm.at[wid], idx_v.at[0])
            gather_col(wtab_hbm, wid * _V, 0, wt_out.at[wid])

    return k(sparse_t3, tab_f, wtab_f)


def _dot0(a, b):
    return lax.dot_general(a, b, (((0,), (0,)), ((), ())),
                           preferred_element_type=jnp.float32)


def _tc_head(embt, da, wt, W1a, W1c, b1s, W2s, b2s, W3, fmW, bias3, M,
             interpret=False):
    grid = (_B // _BM,)

    def full(a):
        return pl.BlockSpec(a.shape, lambda i: tuple(0 for _ in a.shape))

    ones = jnp.ones((_NC, 1), jnp.float32)
    ones26 = jnp.ones((_NF, 1), jnp.float32)

    def body(embt_ref, da_ref, wt_ref, W1a_ref, W1c_ref, b1_ref, W2_ref,
             b2_ref, W3_ref, fmW_ref, b3_ref, M_ref, o416_ref, o26_ref,
             out_ref):
        embt_blk = embt_ref[...]
        da_blk = da_ref[...]
        h = (_dot0(embt_blk, W1a_ref[...])
             + jnp.dot(da_blk, W1c_ref[...],
                       preferred_element_type=jnp.float32)
             + b1_ref[...])
        h = jnp.maximum(h, 0.0)
        h = (jnp.dot(h, W2_ref[...], preferred_element_type=jnp.float32)
             + b2_ref[...])
        h = jnp.maximum(h, 0.0)
        deep = jnp.dot(h, W3_ref[...], preferred_element_type=jnp.float32)
        fm1 = (_dot0(wt_ref[...], o26_ref[...])
               + jnp.dot(da_blk, fmW_ref[...],
                         preferred_element_type=jnp.float32))
        s = _dot0(embt_blk, M_ref[...])
        q = _dot0(embt_blk * embt_blk, o416_ref[...])
        fm2 = 0.5 * (jnp.sum(s * s, axis=1, keepdims=True) - q)
        out_ref[...] = jax.nn.sigmoid(deep + fm1 + fm2 + b3_ref[...])

    return pl.pallas_call(
        body,
        grid=grid,
        in_specs=[
            pl.BlockSpec((_NC, _BM), lambda i: (0, i)),
            pl.BlockSpec((_BM, da.shape[1]), lambda i: (i, 0)),
            pl.BlockSpec((_NF, _BM), lambda i: (0, i)),
            full(W1a), full(W1c), full(b1s), full(W2s), full(b2s),
            full(W3), full(fmW), full(bias3), full(M), full(ones),
            full(ones26),
        ],
        out_specs=pl.BlockSpec((_BM, 1), lambda i: (i, 0)),
        out_shape=jax.ShapeDtypeStruct((_B, 1), jnp.float32),
        interpret=interpret,
    )(embt, da, wt, W1a, W1c, b1s, W2s, b2s, W3, fmW, bias3, M, ones, ones26)


def kernel(dense_inputs, sparse_inputs, bge_inputs, emb_tables, w_tables,
           fm_dense_W, fm_dense_b, W1, b1, g1, be1, W2, b2, g2, be2, W3, b3):
    tab_f = emb_tables.transpose(0, 2, 1).reshape(_NC * _V)
    wtab_f = w_tables.reshape(_NF * _V)
    sparse_t3 = sparse_inputs.astype(jnp.int32).T.reshape(_NF, _JROWS, 128)

    embt, wt = _sc_gather(sparse_t3, tab_f, wtab_f)

    da = jnp.concatenate([dense_inputs, bge_inputs], axis=1)
    inv = 1.0 / jnp.sqrt(jnp.float32(1.0 + 1e-5))
    sc1 = g1 * inv
    W1s = W1 * sc1[None, :]
    b1s = (b1 * sc1 + be1)[None, :]
    sc2 = g2 * inv
    W2s = W2 * sc2[None, :]
    b2s = (b2 * sc2 + be2)[None, :]
    W1a = W1s[: _NC]
    W1c = W1s[_NC:]
    bias3 = (b3 + fm_dense_b).reshape(1, 1)
    M = jnp.tile(jnp.eye(_EMB, dtype=jnp.float32), (_NF, 1))

    return _tc_head(embt, da, wt, W1a, W1c, b1s, W2s, b2s, W3, fmW=fm_dense_W,
                    bias3=bias3, M=M)


# speedup vs baseline: 1.9581x; 1.3512x over previous
import functools

import jax
import jax.numpy as jnp
from jax import lax
from jax.experimental import pallas as pl
from jax.experimental.pallas import tpu as pltpu
from jax.experimental.pallas import tpu_sc as plsc

_B = 16384
_NF = 26
_V = 100000
_EMB = 16
_NC = _NF * _EMB
_NW = 32
_CPW = _NC // _NW
_JROWS = _B // 128
_BM = 512

_MESH = dict(core_axis_name="c", subcore_axis_name="s")


def _fire_col(src_ref, base, idx_row_fn, col_v, buf, sem):
    def grp(s, carry):
        for kk in range(16):
            j = s * 16 + kk
            pltpu.async_copy(
                src_ref.at[pl.ds(base, _V)].at[idx_row_fn(j)],
                col_v.at[buf, pl.ds(j * 128, 128)], sem)
        return carry

    lax.fori_loop(0, _JROWS // 16, grp, 0)


def _drain_col(src_ref, col_v, buf, sem, out_row_ref):
    pltpu.make_async_copy(src_ref.at[pl.ds(0, _B)], col_v.at[buf], sem).wait()
    pltpu.sync_copy(col_v.at[buf], out_row_ref)


def _sc_gather_emb(sparse_t3, tab_f):
    @functools.partial(
        pl.kernel,
        mesh=plsc.VectorSubcoreMesh(**_MESH),
        compiler_params=pltpu.CompilerParams(use_tc_tiling_on_sc=False),
        out_type=jax.ShapeDtypeStruct((_NC, _B), jnp.float32),
        scratch_types=[
            pltpu.VMEM((2, _JROWS, 128), jnp.int32),
            pltpu.VMEM((2, _B), jnp.float32),
            pltpu.SemaphoreType.DMA,
            pltpu.SemaphoreType.DMA,
        ],
    )
    def k(sp_hbm, tab_hbm, embt_out, idx_v, col_v, s0, s1):
        wid = lax.axis_index("s") * 2 + lax.axis_index("c")
        c0 = wid * _CPW
        f0 = lax.shift_right_logical(c0, 4)
        f_last = lax.shift_right_logical(c0 + (_CPW - 1), 4)
        pltpu.sync_copy(sp_hbm.at[f0], idx_v.at[0])
        pltpu.sync_copy(sp_hbm.at[f_last], idx_v.at[1])
        sems = [s0, s1]

        for t in range(_CPW):
            c = c0 + t
            sel = lax.shift_right_logical(c, 4) - f0
            _fire_col(tab_hbm, c * _V, lambda j: idx_v.at[sel, j],
                      col_v, t % 2, sems[t % 2])
            if t:
                _drain_col(tab_hbm, col_v, (t - 1) % 2, sems[(t - 1) % 2],
                           embt_out.at[c - 1])
        _drain_col(tab_hbm, col_v, (_CPW - 1) % 2, sems[(_CPW - 1) % 2],
                   embt_out.at[c0 + _CPW - 1])

    return k(sparse_t3, tab_f)


def _sc_gather_w(sparse_t3, wtab_f):
    @functools.partial(
        pl.kernel,
        mesh=plsc.VectorSubcoreMesh(**_MESH),
        compiler_params=pltpu.CompilerParams(use_tc_tiling_on_sc=False),
        out_type=jax.ShapeDtypeStruct((_NF, _B), jnp.float32),
        scratch_types=[
            pltpu.VMEM((_JROWS, 128), jnp.int32),
            pltpu.VMEM((1, _B), jnp.float32),
            pltpu.SemaphoreType.DMA,
        ],
    )
    def k(sp_hbm, wtab_hbm, wt_out, idx_v, col_v, sem):
        wid = lax.axis_index("s") * 2 + lax.axis_index("c")

        @pl.when(wid < _NF)
        def _():
            pltpu.sync_copy(sp_hbm.at[wid], idx_v)
            _fire_col(wtab_hbm, wid * _V, lambda j: idx_v.at[j],
                      col_v, 0, sem)
            _drain_col(wtab_hbm, col_v, 0, sem, wt_out.at[wid])

    return k(sparse_t3, wtab_f)


def _dot0(a, b):
    return lax.dot_general(a, b, (((0,), (0,)), ((), ())),
                           preferred_element_type=jnp.float32)


def _tc_head(embt, da, wt, W1a, W1c, b1s, W2s, b2s, W3, fmW, bias3, M,
             interpret=False):
    grid = (_B // _BM,)

    def full(a):
        return pl.BlockSpec(a.shape, lambda i: tuple(0 for _ in a.shape))

    ones = jnp.ones((_NC, 1), jnp.float32)
    ones26 = jnp.ones((_NF, 1), jnp.float32)

    def body(embt_ref, da_ref, wt_ref, W1a_ref, W1c_ref, b1_ref, W2_ref,
             b2_ref, W3_ref, fmW_ref, b3_ref, M_ref, o416_ref, o26_ref,
             out_ref):
        embt_blk = embt_ref[...]
        da_blk = da_ref[...]
        h = (_dot0(embt_blk, W1a_ref[...])
             + jnp.dot(da_blk, W1c_ref[...],
                       preferred_element_type=jnp.float32)
             + b1_ref[...])
        h = jnp.maximum(h, 0.0)
        h = (jnp.dot(h, W2_ref[...], preferred_element_type=jnp.float32)
             + b2_ref[...])
        h = jnp.maximum(h, 0.0)
        deep = jnp.dot(h, W3_ref[...], preferred_element_type=jnp.float32)
        fm1 = (_dot0(wt_ref[...], o26_ref[...])
               + jnp.dot(da_blk, fmW_ref[...],
                         preferred_element_type=jnp.float32))
        s = _dot0(embt_blk, M_ref[...])
        q = _dot0(embt_blk * embt_blk, o416_ref[...])
        fm2 = 0.5 * (jnp.sum(s * s, axis=1, keepdims=True) - q)
        out_ref[...] = jax.nn.sigmoid(deep + fm1 + fm2 + b3_ref[...])

    return pl.pallas_call(
        body,
        grid=grid,
        in_specs=[
            pl.BlockSpec((_NC, _BM), lambda i: (0, i)),
            pl.BlockSpec((_BM, da.shape[1]), lambda i: (i, 0)),
            pl.BlockSpec((_NF, _BM), lambda i: (0, i)),
            full(W1a), full(W1c), full(b1s), full(W2s), full(b2s),
            full(W3), full(fmW), full(bias3), full(M), full(ones),
            full(ones26),
        ],
        out_specs=pl.BlockSpec((_BM, 1), lambda i: (i, 0)),
        out_shape=jax.ShapeDtypeStruct((_B, 1), jnp.float32),
        interpret=interpret,
    )(embt, da, wt, W1a, W1c, b1s, W2s, b2s, W3, fmW, bias3, M, ones, ones26)


def kernel(dense_inputs, sparse_inputs, bge_inputs, emb_tables, w_tables,
           fm_dense_W, fm_dense_b, W1, b1, g1, be1, W2, b2, g2, be2, W3, b3):
    tab_f = emb_tables.transpose(0, 2, 1).reshape(_NC * _V)
    wtab_f = w_tables.reshape(_NF * _V)
    sparse_t3 = sparse_inputs.astype(jnp.int32).T.reshape(_NF, _JROWS, 128)

    embt = _sc_gather_emb(sparse_t3, tab_f)
    wt = _sc_gather_w(sparse_t3, wtab_f)

    da = jnp.concatenate([dense_inputs, bge_inputs], axis=1)
    inv = 1.0 / jnp.sqrt(jnp.float32(1.0 + 1e-5))
    sc1 = g1 * inv
    W1s = W1 * sc1[None, :]
    b1s = (b1 * sc1 + be1)[None, :]
    sc2 = g2 * inv
    W2s = W2 * sc2[None, :]
    b2s = (b2 * sc2 + be2)[None, :]
    W1a = W1s[: _NC]
    W1c = W1s[_NC:]
    bias3 = (b3 + fm_dense_b).reshape(1, 1)
    M = jnp.tile(jnp.eye(_EMB, dtype=jnp.float32), (_NF, 1))

    return _tc_head(embt, da, wt, W1a, W1c, b1s, W2s, b2s, W3, fmW=fm_dense_W,
                    bias3=bias3, M=M)
